# baseline (speedup 1.0000x reference)
import jax
import jax.numpy as jnp
from jax import lax
from jax.experimental import pallas as pl
from jax.experimental.pallas import tpu as pltpu
from jax.experimental.pallas import tpu_sc as plsc

B, T = 16, 750
DW, DL, DM = 1280, 1024, 1024
D = DW + DL + DM
CODE_DIM, K = 256, 1024
N = B * T

G = 4
GB = B // G

WIN = 8
NWK = 8
WSPAN = 96
TLIM = 744
NWIN = 12

_mesh = plsc.VectorSubcoreMesh(core_axis_name="core", subcore_axis_name="subcore")


def _dec_body(cb_ref, wd_ref, bd_ref, out_ref):
    out_ref[...] = (jnp.dot(cb_ref[...].astype(jnp.bfloat16), wd_ref[...],
                            preferred_element_type=jnp.float32)
                    + bd_ref[...])


def _main_body(w_ref, l_ref, m_ref, wew_ref, wel_ref, wem_ref, be_ref,
               cbt_ref, cbb_ref, wd_ref, bd_ref,
               codes_ref, tail_ref, ps_ref, cs_ref,
               acc_ref, csum_ref):
    i = pl.program_id(0)

    @pl.when(i == 0)
    def _init():
        acc_ref[...] = jnp.zeros_like(acc_ref)
        cbt = cbt_ref[...]
        acc_ref[1:2, :] = jnp.sum(cbt * cbt, axis=0, keepdims=True)
        csum_ref[...] = jnp.zeros_like(csum_ref)

    ze = (jnp.dot(w_ref[0], wew_ref[...], preferred_element_type=jnp.float32)
          + jnp.dot(l_ref[0], wel_ref[...], preferred_element_type=jnp.float32)
          + jnp.dot(m_ref[0], wem_ref[...], preferred_element_type=jnp.float32)
          + be_ref[...])

    z2 = jnp.sum(ze * ze, axis=1, keepdims=True)
    zc = jnp.dot(ze, cbt_ref[...], preferred_element_type=jnp.float32)
    d = z2 - 2.0 * zc + acc_ref[1:2, :]

    dmin = jnp.min(d, axis=1, keepdims=True)
    kiota = jax.lax.broadcasted_iota(jnp.int32, d.shape, 1)
    codes = jnp.min(jnp.where(d == dmin, kiota, K), axis=1,
                    keepdims=True)
    codes_ref[0] = codes

    p = jnp.exp(dmin - d)
    s = jnp.sum(p, axis=1, keepdims=True)
    acc_ref[0:1, :] = acc_ref[0:1, :] + jnp.sum(p / s, axis=0, keepdims=True)
    csum_ref[0:1, 0:1] = csum_ref[0:1, 0:1] + jnp.sum(dmin, axis=0,
                                                      keepdims=True)

    ct = jnp.concatenate(
        [lax.slice(codes, (TLIM, 0), (T, 1)),
         jnp.zeros((2, 1), jnp.int32)], axis=0)
    kiota8 = jax.lax.broadcasted_iota(jnp.int32, (8, K), 1)
    oneh = (kiota8 == ct).astype(jnp.bfloat16)
    zqt = jnp.dot(oneh, cbb_ref[...], preferred_element_type=jnp.float32)
    ze_t = jnp.concatenate(
        [lax.slice(ze, (TLIM, 0), (T, CODE_DIM)),
         jnp.zeros((2, CODE_DIM), jnp.float32)], axis=0)
    zq_st = ze_t + (zqt - ze_t)
    tail_ref[0] = (jnp.dot(zq_st.astype(jnp.bfloat16), wd_ref[...],
                           preferred_element_type=jnp.float32)
                   + bd_ref[...])

    @pl.when(i == B - 1)
    def _fin():
        ps_ref[...] = acc_ref[0:1, :]
        cs_ref[...] = csum_ref[0:1, 0:1]


def _make_sc_gather(g, first):
    def body(tab_hbm, codes_hbm, o_hbm, idx_v, b0, b1, b2, b3,
             s0, s1, s2, s3, t0s, t1s, t2s, t3s):
        wid = lax.axis_index("core") * 16 + lax.axis_index("subcore")
        bl = wid // NWK
        h = wid % NWK
        b = g * GB + bl
        base = h * WSPAN

        pltpu.sync_copy(codes_hbm.at[bl, 0], idx_v)

        bufs = (b0, b1, b2, b3)
        gsems = (s0, s1, s2, s3)
        osems = (t0s, t1s, t2s, t3s)

        def t0c(w):
            return pl.multiple_of(jnp.minimum(base + w * WIN, TLIM - WIN), 8)

        def gidx(w):
            return idx_v.at[pl.ds(t0c(w), WIN)]

        def oslc(w):
            return o_hbm.at[b, pl.ds(t0c(w), WIN), :]

        def gstart(w, k):
            pltpu.make_async_copy(tab_hbm.at[gidx(w)], bufs[k],
                                  gsems[k]).start()

        def gwait(w, k):
            pltpu.make_async_copy(tab_hbm.at[gidx(w)], bufs[k],
                                  gsems[k]).wait()

        def ostart(w, k):
            pltpu.make_async_copy(bufs[k], oslc(w), osems[k]).start()

        def owait(w, k):
            pltpu.make_async_copy(bufs[k], oslc(w), osems[k]).wait()

        gstart(0, 0)
        gstart(1, 1)

        @pl.loop(0, NWIN // 4)
        def _(m):
            w = m * 4
            gwait(w, 0)
            gwait(w + 1, 1)

            @pl.when(m > 0)
            def _():
                owait(w - 2, 2)
                owait(w - 1, 3)

            gstart(w + 2, 2)
            gstart(w + 3, 3)
            ostart(w, 0)
            ostart(w + 1, 1)

            gwait(w + 2, 2)
            gwait(w + 3, 3)
            owait(w, 0)
            owait(w + 1, 1)
            gstart(w + 4, 0)
            gstart(w + 5, 1)
            ostart(w + 2, 2)
            ostart(w + 3, 3)

        gwait(NWIN, 0)
        gwait(NWIN + 1, 1)
        owait(NWIN - 2, 2)
        owait(NWIN - 1, 3)

    scratch = [
        pltpu.VMEM((T,), jnp.int32),
        pltpu.VMEM((WIN, D), jnp.float32),
        pltpu.VMEM((WIN, D), jnp.float32),
        pltpu.VMEM((WIN, D), jnp.float32),
        pltpu.VMEM((WIN, D), jnp.float32),
    ] + [pltpu.SemaphoreType.DMA] * 8

    if first:
        return pl.kernel(body, out_type=jax.ShapeDtypeStruct((B, T, D),
                                                             jnp.float32),
                         mesh=_mesh, scratch_types=scratch)

    def body_ref(tab_hbm, codes_hbm, o_ref):
        return body(tab_hbm, codes_hbm, o_ref)

    return pl.kernel(body, out_type=(), mesh=_mesh, scratch_types=scratch)


def _final_body(dummy_ref, tail_ref, ps_ref, cs_ref,
                out_ref, commit_ref, ent_ref):
    i = pl.program_id(0)
    out_ref[...] = tail_ref[...]

    @pl.when(i == B - 1)
    def _fin():
        commit_ref[...] = cs_ref[...] / (N * CODE_DIM)
        avg = ps_ref[...] / N
        ent_ref[...] = jnp.sum(avg * jnp.log(avg + 1e-10), axis=1,
                               keepdims=True)


@jax.jit
def kernel(whisper_feat, wavlm_feat, muq_feat, W_enc, b_enc, codebook,
           W_dec, b_dec):
    wew = W_enc[:DW]
    wel = W_enc[DW:DW + DL]
    wem = W_enc[DW + DL:]
    cbt = codebook.T
    cbb = codebook.astype(jnp.bfloat16)
    wdb = W_dec.astype(jnp.bfloat16)
    be2 = b_enc.reshape(1, CODE_DIM)
    bd2 = b_dec.reshape(1, D)

    dec_cb = pl.pallas_call(
        _dec_body,
        out_shape=jax.ShapeDtypeStruct((K, D), jnp.float32),
    )(codebook, wdb, bd2)

    def run_main():
        return pl.pallas_call(
            _main_body,
            grid=(B,),
            in_specs=[
                pl.BlockSpec((1, T, DW), lambda i: (i, 0, 0)),
                pl.BlockSpec((1, T, DL), lambda i: (i, 0, 0)),
                pl.BlockSpec((1, T, DM), lambda i: (i, 0, 0)),
                pl.BlockSpec((DW, CODE_DIM), lambda i: (0, 0)),
                pl.BlockSpec((DL, CODE_DIM), lambda i: (0, 0)),
                pl.BlockSpec((DM, CODE_DIM), lambda i: (0, 0)),
                pl.BlockSpec((1, CODE_DIM), lambda i: (0, 0)),
                pl.BlockSpec((CODE_DIM, K), lambda i: (0, 0)),
                pl.BlockSpec((K, CODE_DIM), lambda i: (0, 0)),
                pl.BlockSpec((CODE_DIM, D), lambda i: (0, 0)),
                pl.BlockSpec((1, D), lambda i: (0, 0)),
            ],
            out_specs=[
                pl.BlockSpec((1, T, 1), lambda i: (i, 0, 0)),
                pl.BlockSpec((1, 8, D), lambda i: (i, 0, 0)),
                pl.BlockSpec((1, K), lambda i: (0, 0)),
                pl.BlockSpec((1, 1), lambda i: (0, 0)),
            ],
            out_shape=[
                jax.ShapeDtypeStruct((B, T, 1), jnp.int32),
                jax.ShapeDtypeStruct((B, 8, D), jnp.float32),
                jax.ShapeDtypeStruct((1, K), jnp.float32),
                jax.ShapeDtypeStruct((1, 1), jnp.float32),
            ],
            scratch_shapes=[
                pltpu.VMEM((8, K), jnp.float32),
                pltpu.VMEM((8, 128), jnp.float32),
            ],
        )(whisper_feat, wavlm_feat, muq_feat, wew, wel, wem, be2, cbt, cbb,
          wdb, bd2)

    codes_c, tails, ps, cs = run_main()

    recon_ref = None
    for g in range(G):
        cg = lax.slice(codes_c, (g * GB, 0, 0),
                       (g * GB + GB, T, 1)).reshape(GB, 1, T)
        if g == 0:
            recon0 = _make_sc_gather(0, True)(dec_cb, cg)
            recon_ref = jax.new_ref(recon0)
        else:
            _make_sc_gather(g, False)(dec_cb, cg, recon_ref)
    recon_sc = recon_ref[...]

    recon, commit, ent = pl.pallas_call(
        _final_body,
        grid=(B,),
        in_specs=[
            pl.BlockSpec((1, 8, 128), lambda i: (i, 93, 0)),
            pl.BlockSpec((1, 8, D), lambda i: (i, 0, 0)),
            pl.BlockSpec((1, K), lambda i: (0, 0)),
            pl.BlockSpec((1, 1), lambda i: (0, 0)),
        ],
        out_specs=[
            pl.BlockSpec((1, 8, D), lambda i: (i, 93, 0)),
            pl.BlockSpec((1, 1), lambda i: (0, 0)),
            pl.BlockSpec((1, 1), lambda i: (0, 0)),
        ],
        out_shape=[
            jax.ShapeDtypeStruct((B, T, D), jnp.float32),
            jax.ShapeDtypeStruct((1, 1), jnp.float32),
            jax.ShapeDtypeStruct((1, 1), jnp.float32),
        ],
        input_output_aliases={0: 0},
    )(recon_sc, tails, ps, cs)

    codes = codes_c.reshape(B, T)
    return (recon, codes, commit[0, 0], None, ent[0, 0])


# speedup vs baseline: 1.0596x; 1.0596x over previous
import jax
import jax.numpy as jnp
from jax import lax
from jax.experimental import pallas as pl
from jax.experimental.pallas import tpu as pltpu
from jax.experimental.pallas import tpu_sc as plsc

B, T = 16, 750
DW, DL, DM = 1280, 1024, 1024
D = DW + DL + DM
CODE_DIM, K = 256, 1024
N = B * T

G = 4
GB = B // G

WIN = 8
NWK = 8
WSPAN = 96
TLIM = 744
NWIN = 12

_mesh = plsc.VectorSubcoreMesh(core_axis_name="core", subcore_axis_name="subcore")


def _dec_body(cb_ref, wd_ref, bd_ref, out_ref):
    out_ref[...] = (jnp.dot(cb_ref[...].astype(jnp.bfloat16), wd_ref[...],
                            preferred_element_type=jnp.float32)
                    + bd_ref[...])


def _main_body(w_ref, l_ref, m_ref, wew_ref, wel_ref, wem_ref, be_ref,
               cbt_ref, cbb_ref, wd_ref, bd_ref,
               codes_ref, tail_ref, ps_ref, cs_ref,
               acc_ref, csum_ref):
    i = pl.program_id(0)

    @pl.when(i == 0)
    def _init():
        acc_ref[...] = jnp.zeros_like(acc_ref)
        cbt = cbt_ref[...]
        acc_ref[1:2, :] = jnp.sum(cbt * cbt, axis=0, keepdims=True)
        csum_ref[...] = jnp.zeros_like(csum_ref)

    ze = (jnp.dot(w_ref[0], wew_ref[...], preferred_element_type=jnp.float32)
          + jnp.dot(l_ref[0], wel_ref[...], preferred_element_type=jnp.float32)
          + jnp.dot(m_ref[0], wem_ref[...], preferred_element_type=jnp.float32)
          + be_ref[...])

    z2 = jnp.sum(ze * ze, axis=1, keepdims=True)
    zc = jnp.dot(ze, cbt_ref[...], preferred_element_type=jnp.float32)
    d = z2 - 2.0 * zc + acc_ref[1:2, :]

    dmin = jnp.min(d, axis=1, keepdims=True)
    kiota = jax.lax.broadcasted_iota(jnp.int32, d.shape, 1)
    codes = jnp.min(jnp.where(d == dmin, kiota, K), axis=1,
                    keepdims=True)
    codes_ref[0] = codes

    p = jnp.exp(dmin - d)
    s = jnp.sum(p, axis=1, keepdims=True)
    acc_ref[0:1, :] = acc_ref[0:1, :] + jnp.sum(p / s, axis=0, keepdims=True)
    csum_ref[0:1, 0:1] = csum_ref[0:1, 0:1] + jnp.sum(dmin, axis=0,
                                                      keepdims=True)

    ct = jnp.concatenate(
        [lax.slice(codes, (TLIM, 0), (T, 1)),
         jnp.zeros((2, 1), jnp.int32)], axis=0)
    kiota8 = jax.lax.broadcasted_iota(jnp.int32, (8, K), 1)
    oneh = (kiota8 == ct).astype(jnp.bfloat16)
    zqt = jnp.dot(oneh, cbb_ref[...], preferred_element_type=jnp.float32)
    ze_t = jnp.concatenate(
        [lax.slice(ze, (TLIM, 0), (T, CODE_DIM)),
         jnp.zeros((2, CODE_DIM), jnp.float32)], axis=0)
    zq_st = ze_t + (zqt - ze_t)
    tail_ref[0] = (jnp.dot(zq_st.astype(jnp.bfloat16), wd_ref[...],
                           preferred_element_type=jnp.float32)
                   + bd_ref[...])

    @pl.when(i == GB - 1)
    def _fin():
        ps_ref[...] = acc_ref[0:1, :]
        cs_ref[...] = csum_ref[0:1, 0:1]


def _make_sc_gather(g, first):
    def body(tab_hbm, codes_hbm, o_hbm, idx_v, b0, b1, b2, b3,
             s0, s1, s2, s3, t0s, t1s, t2s, t3s):
        wid = lax.axis_index("core") * 16 + lax.axis_index("subcore")
        bl = wid // NWK
        h = wid % NWK
        b = g * GB + bl
        base = h * WSPAN

        pltpu.sync_copy(codes_hbm.at[bl, 0], idx_v)

        bufs = (b0, b1, b2, b3)
        gsems = (s0, s1, s2, s3)
        osems = (t0s, t1s, t2s, t3s)

        def t0c(w):
            return pl.multiple_of(jnp.minimum(base + w * WIN, TLIM - WIN), 8)

        def gidx(w):
            return idx_v.at[pl.ds(t0c(w), WIN)]

        def oslc(w):
            return o_hbm.at[b, pl.ds(t0c(w), WIN), :]

        def gstart(w, k):
            pltpu.make_async_copy(tab_hbm.at[gidx(w)], bufs[k],
                                  gsems[k]).start()

        def gwait(w, k):
            pltpu.make_async_copy(tab_hbm.at[gidx(w)], bufs[k],
                                  gsems[k]).wait()

        def ostart(w, k):
            pltpu.make_async_copy(bufs[k], oslc(w), osems[k]).start()

        def owait(w, k):
            pltpu.make_async_copy(bufs[k], oslc(w), osems[k]).wait()

        gstart(0, 0)
        gstart(1, 1)

        @pl.loop(0, NWIN // 4)
        def _(m):
            w = m * 4
            gwait(w, 0)
            gwait(w + 1, 1)

            @pl.when(m > 0)
            def _():
                owait(w - 2, 2)
                owait(w - 1, 3)

            gstart(w + 2, 2)
            gstart(w + 3, 3)
            ostart(w, 0)
            ostart(w + 1, 1)

            gwait(w + 2, 2)
            gwait(w + 3, 3)
            owait(w, 0)
            owait(w + 1, 1)
            gstart(w + 4, 0)
            gstart(w + 5, 1)
            ostart(w + 2, 2)
            ostart(w + 3, 3)

        gwait(NWIN, 0)
        gwait(NWIN + 1, 1)
        owait(NWIN - 2, 2)
        owait(NWIN - 1, 3)

    scratch = [
        pltpu.VMEM((T,), jnp.int32),
        pltpu.VMEM((WIN, D), jnp.float32),
        pltpu.VMEM((WIN, D), jnp.float32),
        pltpu.VMEM((WIN, D), jnp.float32),
        pltpu.VMEM((WIN, D), jnp.float32),
    ] + [pltpu.SemaphoreType.DMA] * 8

    if first:
        return pl.kernel(body, out_type=jax.ShapeDtypeStruct((B, T, D),
                                                             jnp.float32),
                         mesh=_mesh, scratch_types=scratch)

    def body_ref(tab_hbm, codes_hbm, o_ref):
        return body(tab_hbm, codes_hbm, o_ref)

    return pl.kernel(body, out_type=(), mesh=_mesh, scratch_types=scratch)


def _tail_body(dummy_ref, tail_ref, out_ref):
    out_ref[...] = tail_ref[...]


def _stats_body(ps_ref, cs_ref, commit_ref, ent_ref):
    commit_ref[...] = jnp.sum(cs_ref[...], axis=0,
                              keepdims=True) / (N * CODE_DIM)
    avg = jnp.sum(ps_ref[...], axis=0, keepdims=True) / N
    ent_ref[...] = jnp.sum(avg * jnp.log(avg + 1e-10), axis=1, keepdims=True)


@jax.jit
def kernel(whisper_feat, wavlm_feat, muq_feat, W_enc, b_enc, codebook,
           W_dec, b_dec):
    wew = W_enc[:DW]
    wel = W_enc[DW:DW + DL]
    wem = W_enc[DW + DL:]
    cbt = codebook.T
    cbb = codebook.astype(jnp.bfloat16)
    wdb = W_dec.astype(jnp.bfloat16)
    be2 = b_enc.reshape(1, CODE_DIM)
    bd2 = b_dec.reshape(1, D)

    dec_cb = pl.pallas_call(
        _dec_body,
        out_shape=jax.ShapeDtypeStruct((K, D), jnp.float32),
    )(codebook, wdb, bd2)

    def run_main(g):
        return pl.pallas_call(
            _main_body,
            grid=(GB,),
            in_specs=[
                pl.BlockSpec((1, T, DW), lambda i: (g * GB + i, 0, 0)),
                pl.BlockSpec((1, T, DL), lambda i: (g * GB + i, 0, 0)),
                pl.BlockSpec((1, T, DM), lambda i: (g * GB + i, 0, 0)),
                pl.BlockSpec((DW, CODE_DIM), lambda i: (0, 0)),
                pl.BlockSpec((DL, CODE_DIM), lambda i: (0, 0)),
                pl.BlockSpec((DM, CODE_DIM), lambda i: (0, 0)),
                pl.BlockSpec((1, CODE_DIM), lambda i: (0, 0)),
                pl.BlockSpec((CODE_DIM, K), lambda i: (0, 0)),
                pl.BlockSpec((K, CODE_DIM), lambda i: (0, 0)),
                pl.BlockSpec((CODE_DIM, D), lambda i: (0, 0)),
                pl.BlockSpec((1, D), lambda i: (0, 0)),
            ],
            out_specs=[
                pl.BlockSpec((1, T, 1), lambda i: (i, 0, 0)),
                pl.BlockSpec((1, 8, D), lambda i: (i, 0, 0)),
                pl.BlockSpec((1, K), lambda i: (0, 0)),
                pl.BlockSpec((1, 1), lambda i: (0, 0)),
            ],
            out_shape=[
                jax.ShapeDtypeStruct((GB, T, 1), jnp.int32),
                jax.ShapeDtypeStruct((GB, 8, D), jnp.float32),
                jax.ShapeDtypeStruct((1, K), jnp.float32),
                jax.ShapeDtypeStruct((1, 1), jnp.float32),
            ],
            scratch_shapes=[
                pltpu.VMEM((8, K), jnp.float32),
                pltpu.VMEM((8, 128), jnp.float32),
            ],
        )(whisper_feat, wavlm_feat, muq_feat, wew, wel, wem, be2, cbt, cbb,
          wdb, bd2)

    codes_l, tails_l, ps_l, cs_l = [], [], [], []
    recon_ref = None
    for g in range(G):
        c, t, p, s = run_main(g)
        codes_l.append(c)
        tails_l.append(t)
        ps_l.append(p)
        cs_l.append(s)
        if g == 0:
            recon0 = _make_sc_gather(0, True)(dec_cb, c.reshape(GB, 1, T))
            recon_ref = jax.new_ref(recon0)
        else:
            _make_sc_gather(g, False)(dec_cb, c.reshape(GB, 1, T),
                                      recon_ref)
    recon_sc = recon_ref[...]

    tails = jnp.concatenate(tails_l, axis=0)
    recon = pl.pallas_call(
        _tail_body,
        grid=(B,),
        in_specs=[
            pl.BlockSpec((1, 8, 128), lambda i: (i, 93, 0)),
            pl.BlockSpec((1, 8, D), lambda i: (i, 0, 0)),
        ],
        out_specs=pl.BlockSpec((1, 8, D), lambda i: (i, 93, 0)),
        out_shape=jax.ShapeDtypeStruct((B, T, D), jnp.float32),
        input_output_aliases={0: 0},
    )(recon_sc, tails)

    commit, ent = pl.pallas_call(
        _stats_body,
        out_shape=[
            jax.ShapeDtypeStruct((1, 1), jnp.float32),
            jax.ShapeDtypeStruct((1, 1), jnp.float32),
        ],
    )(jnp.concatenate(ps_l, axis=0), jnp.concatenate(cs_l, axis=0))

    codes = jnp.concatenate(codes_l, axis=0).reshape(B, T)
    return (recon, codes, commit[0, 0], None, ent[0, 0])


# speedup vs baseline: 1.0611x; 1.0015x over previous
import jax
import jax.numpy as jnp
from jax import lax
from jax.experimental import pallas as pl
from jax.experimental.pallas import tpu as pltpu
from jax.experimental.pallas import tpu_sc as plsc

B, T = 16, 750
DW, DL, DM = 1280, 1024, 1024
D = DW + DL + DM
CODE_DIM, K = 256, 1024
N = B * T

G = 4
GB = B // G

WIN = 8
NWK = 8
WSPAN = 96
TLIM = 744
NWIN = 12

_mesh = plsc.VectorSubcoreMesh(core_axis_name="core", subcore_axis_name="subcore")


def _dec_body(cb_ref, wd_ref, bd_ref, out_ref):
    out_ref[...] = (jnp.dot(cb_ref[...].astype(jnp.bfloat16), wd_ref[...],
                            preferred_element_type=jnp.float32)
                    + bd_ref[...])


def _main_body(w_ref, l_ref, m_ref, wew_ref, wel_ref, wem_ref, be_ref,
               cbt_ref, cbb_ref, wd_ref, bd_ref,
               codes_ref, tail_ref, ps_ref, cs_ref,
               acc_ref, csum_ref):
    i = pl.program_id(0)

    @pl.when(i == 0)
    def _init():
        acc_ref[...] = jnp.zeros_like(acc_ref)
        cbt = cbt_ref[...]
        acc_ref[1:2, :] = jnp.sum(cbt * cbt, axis=0, keepdims=True)
        csum_ref[...] = jnp.zeros_like(csum_ref)

    ze = (jnp.dot(w_ref[0], wew_ref[...], preferred_element_type=jnp.float32)
          + jnp.dot(l_ref[0], wel_ref[...], preferred_element_type=jnp.float32)
          + jnp.dot(m_ref[0], wem_ref[...], preferred_element_type=jnp.float32)
          + be_ref[...])

    z2 = jnp.sum(ze * ze, axis=1, keepdims=True)
    zc = jnp.dot(ze, cbt_ref[...], preferred_element_type=jnp.float32)
    d = z2 - 2.0 * zc + acc_ref[1:2, :]

    dmin = jnp.min(d, axis=1, keepdims=True)
    kiota = jax.lax.broadcasted_iota(jnp.int32, d.shape, 1)
    codes = jnp.min(jnp.where(d == dmin, kiota, K), axis=1,
                    keepdims=True)
    codes_ref[0] = codes

    p = jnp.exp(dmin - d)
    s = jnp.sum(p, axis=1, keepdims=True)
    acc_ref[0:1, :] = acc_ref[0:1, :] + jnp.sum(p / s, axis=0, keepdims=True)
    csum_ref[0:1, 0:1] = csum_ref[0:1, 0:1] + jnp.sum(dmin, axis=0,
                                                      keepdims=True)

    ct = jnp.concatenate(
        [lax.slice(codes, (TLIM, 0), (T, 1)),
         jnp.zeros((2, 1), jnp.int32)], axis=0)
    kiota8 = jax.lax.broadcasted_iota(jnp.int32, (8, K), 1)
    oneh = (kiota8 == ct).astype(jnp.bfloat16)
    zqt = jnp.dot(oneh, cbb_ref[...], preferred_element_type=jnp.float32)
    ze_t = jnp.concatenate(
        [lax.slice(ze, (TLIM, 0), (T, CODE_DIM)),
         jnp.zeros((2, CODE_DIM), jnp.float32)], axis=0)
    zq_st = ze_t + (zqt - ze_t)
    tail_ref[0] = (jnp.dot(zq_st.astype(jnp.bfloat16), wd_ref[...],
                           preferred_element_type=jnp.float32)
                   + bd_ref[...])

    @pl.when(i == GB - 1)
    def _fin():
        ps_ref[...] = acc_ref[0:1, :]
        cs_ref[...] = csum_ref[0:1, 0:1]


def _make_sc_gather(g, first):
    def body(tab_hbm, codes_hbm, o_hbm, idx_v, b0, b1, b2, b3,
             s0, s1, s2, s3, t0s, t1s, t2s, t3s):
        wid = lax.axis_index("core") * 16 + lax.axis_index("subcore")
        bl = wid // NWK
        h = wid % NWK
        b = g * GB + bl
        base = h * WSPAN

        pltpu.sync_copy(codes_hbm.at[bl, 0], idx_v)

        bufs = (b0, b1, b2, b3)
        gsems = (s0, s1, s2, s3)
        osems = (t0s, t1s, t2s, t3s)

        def t0c(w):
            return pl.multiple_of(jnp.minimum(base + w * WIN, TLIM - WIN), 8)

        def gidx(w):
            return idx_v.at[pl.ds(t0c(w), WIN)]

        def oslc(w):
            return o_hbm.at[b, pl.ds(t0c(w), WIN), :]

        def gstart(w, k):
            pltpu.make_async_copy(tab_hbm.at[gidx(w)], bufs[k],
                                  gsems[k]).start()

        def gwait(w, k):
            pltpu.make_async_copy(tab_hbm.at[gidx(w)], bufs[k],
                                  gsems[k]).wait()

        def ostart(w, k):
            pltpu.make_async_copy(bufs[k], oslc(w), osems[k]).start()

        def owait(w, k):
            pltpu.make_async_copy(bufs[k], oslc(w), osems[k]).wait()

        gstart(0, 0)
        gstart(1, 1)

        @pl.loop(0, NWIN // 4)
        def _(m):
            w = m * 4
            gwait(w, 0)
            gwait(w + 1, 1)

            @pl.when(m > 0)
            def _():
                owait(w - 2, 2)
                owait(w - 1, 3)

            gstart(w + 2, 2)
            gstart(w + 3, 3)
            ostart(w, 0)
            ostart(w + 1, 1)

            gwait(w + 2, 2)
            gwait(w + 3, 3)
            owait(w, 0)
            owait(w + 1, 1)
            gstart(w + 4, 0)
            gstart(w + 5, 1)
            ostart(w + 2, 2)
            ostart(w + 3, 3)

        gwait(NWIN, 0)
        gwait(NWIN + 1, 1)
        owait(NWIN - 2, 2)
        owait(NWIN - 1, 3)

    scratch = [
        pltpu.VMEM((T,), jnp.int32),
        pltpu.VMEM((WIN, D), jnp.float32),
        pltpu.VMEM((WIN, D), jnp.float32),
        pltpu.VMEM((WIN, D), jnp.float32),
        pltpu.VMEM((WIN, D), jnp.float32),
    ] + [pltpu.SemaphoreType.DMA] * 8

    if first:
        return pl.kernel(body, out_type=jax.ShapeDtypeStruct((B, T, D),
                                                             jnp.float32),
                         mesh=_mesh, scratch_types=scratch)

    return pl.kernel(body, out_type=(), mesh=_mesh, scratch_types=scratch)


def _tail_body(dummy_ref, tail_ref, out_ref):
    out_ref[...] = tail_ref[...]


def _stats_body(ps_ref, cs_ref, commit_ref, ent_ref):
    commit_ref[...] = jnp.sum(cs_ref[...], axis=0,
                              keepdims=True) / (N * CODE_DIM)
    avg = jnp.sum(ps_ref[...], axis=0, keepdims=True) / N
    ent_ref[...] = jnp.sum(avg * jnp.log(avg + 1e-10), axis=1, keepdims=True)


@jax.jit
def kernel(whisper_feat, wavlm_feat, muq_feat, W_enc, b_enc, codebook,
           W_dec, b_dec):
    wew = W_enc[:DW]
    wel = W_enc[DW:DW + DL]
    wem = W_enc[DW + DL:]
    cbt = codebook.T
    cbb = codebook.astype(jnp.bfloat16)
    wdb = W_dec.astype(jnp.bfloat16)
    be2 = b_enc.reshape(1, CODE_DIM)
    bd2 = b_dec.reshape(1, D)

    dec_cb = pl.pallas_call(
        _dec_body,
        out_shape=jax.ShapeDtypeStruct((K, D), jnp.float32),
    )(codebook, wdb, bd2)

    def run_main(g):
        return pl.pallas_call(
            _main_body,
            grid=(GB,),
            in_specs=[
                pl.BlockSpec((1, T, DW), lambda i: (g * GB + i, 0, 0)),
                pl.BlockSpec((1, T, DL), lambda i: (g * GB + i, 0, 0)),
                pl.BlockSpec((1, T, DM), lambda i: (g * GB + i, 0, 0)),
                pl.BlockSpec((DW, CODE_DIM), lambda i: (0, 0)),
                pl.BlockSpec((DL, CODE_DIM), lambda i: (0, 0)),
                pl.BlockSpec((DM, CODE_DIM), lambda i: (0, 0)),
                pl.BlockSpec((1, CODE_DIM), lambda i: (0, 0)),
                pl.BlockSpec((CODE_DIM, K), lambda i: (0, 0)),
                pl.BlockSpec((K, CODE_DIM), lambda i: (0, 0)),
                pl.BlockSpec((CODE_DIM, D), lambda i: (0, 0)),
                pl.BlockSpec((1, D), lambda i: (0, 0)),
            ],
            out_specs=[
                pl.BlockSpec((1, T, 1), lambda i: (i, 0, 0)),
                pl.BlockSpec((1, 8, D), lambda i: (i, 0, 0)),
                pl.BlockSpec((1, K), lambda i: (0, 0)),
                pl.BlockSpec((1, 1), lambda i: (0, 0)),
            ],
            out_shape=[
                jax.ShapeDtypeStruct((GB, T, 1), jnp.int32),
                jax.ShapeDtypeStruct((GB, 8, D), jnp.float32),
                jax.ShapeDtypeStruct((1, K), jnp.float32),
                jax.ShapeDtypeStruct((1, 1), jnp.float32),
            ],
            scratch_shapes=[
                pltpu.VMEM((8, K), jnp.float32),
                pltpu.VMEM((8, 128), jnp.float32),
            ],
        )(whisper_feat, wavlm_feat, muq_feat, wew, wel, wem, be2, cbt, cbb,
          wdb, bd2)

    codes_l, tails_l, ps_l, cs_l = [], [], [], []
    recon_ref = None
    for g in range(G):
        c, t, p, s = run_main(g)
        codes_l.append(c)
        tails_l.append(t)
        ps_l.append(p)
        cs_l.append(s)
        if g == 0:
            recon0 = _make_sc_gather(0, True)(dec_cb, c.reshape(GB, 1, T))
            recon_ref = jax.new_ref(recon0)
        else:
            _make_sc_gather(g, False)(dec_cb, c.reshape(GB, 1, T),
                                      recon_ref)
    recon_sc = recon_ref[...]

    tails = jnp.concatenate(tails_l, axis=0)
    recon = pl.pallas_call(
        _tail_body,
        grid=(B,),
        in_specs=[
            pl.BlockSpec((1, 8, 128), lambda i: (i, 93, 0)),
            pl.BlockSpec((1, 8, D), lambda i: (i, 0, 0)),
        ],
        out_specs=pl.BlockSpec((1, 8, D), lambda i: (i, 93, 0)),
        out_shape=jax.ShapeDtypeStruct((B, T, D), jnp.float32),
        input_output_aliases={0: 0},
    )(recon_sc, tails)

    commit, ent = pl.pallas_call(
        _stats_body,
        out_shape=[
            jax.ShapeDtypeStruct((1, 1), jnp.float32),
            jax.ShapeDtypeStruct((1, 1), jnp.float32),
        ],
    )(jnp.concatenate(ps_l, axis=0), jnp.concatenate(cs_l, axis=0))

    codes = jnp.concatenate(codes_l, axis=0).reshape(B, T)
    return (recon, codes, commit[0, 0], None, ent[0, 0])
